# baseline (speedup 1.0000x reference)
import functools
import numpy as np
import jax
import jax.numpy as jnp
from jax import lax
from jax.experimental import pallas as pl
from jax.experimental.pallas import tpu as pltpu
from jax.experimental.pallas import tpu_sc as plsc

N_NODES = 50000
N_CP = 17
N_ENTRIES = 1000000
BATCH_NODES = 4096
PENALTY = 10.0
TIME_MAX = 100.0

NW = 32
CHUNK = 2048
NCHUNK = 16
EV_PER_W = CHUNK * NCHUNK
E_PAD = NW * EV_PER_W
NODES_PER_W = BATCH_NODES // NW
T4_W = 8
TROWS = 8
TN_W = 40

_CP = (np.arange(N_CP, dtype=np.float32) / np.float32(N_CP - 1)) * np.float32(TIME_MAX + 0.0001)
_SEG = float(_CP[1] - _CP[0])

P_LOG, P_PR1, P_PR2 = 0, 1, 2
P_S0, P_S1, P_PQ, P_PC = 4, 21, 38, 55
P_H = 72


def _perm_matrix():
    P = np.zeros((2 * N_CP, TROWS * 8), dtype=np.float32)
    for j in range(TROWS):
        for c in range(8):
            t = 2 * j + c // 2
            if t <= N_CP - 1:
                src_col = t if c % 2 == 0 else N_CP + t
                P[src_col, 8 * j + c] = 1.0
    return P


def _prep_body(z_ref, p_ref, t4_ref):
    ze = jnp.exp(z_ref[...])
    mask = jnp.int32(-65536)
    b = lax.bitcast_convert_type(ze, jnp.int32)
    hi = lax.bitcast_convert_type(jnp.bitwise_and(b, mask), jnp.float32)
    r = ze - hi
    rb = lax.bitcast_convert_type(r, jnp.int32)
    mid = lax.bitcast_convert_type(jnp.bitwise_and(rb, mask), jnp.float32)
    pb = p_ref[...].astype(jnp.bfloat16)
    acc = jnp.dot(hi.astype(jnp.bfloat16), pb, preferred_element_type=jnp.float32)
    acc = acc + jnp.dot(mid.astype(jnp.bfloat16), pb, preferred_element_type=jnp.float32)
    t4_ref[...] = acc


def _prep(zf, perm):
    blk = 25000
    grid = N_NODES // blk
    return pl.pallas_call(
        _prep_body,
        grid=(grid,),
        in_specs=[
            pl.BlockSpec((blk, 2 * N_CP), lambda i: (i, 0)),
            pl.BlockSpec((2 * N_CP, TROWS * 8), lambda i: (0, 0)),
        ],
        out_specs=pl.BlockSpec((blk, TROWS * 8), lambda i: (i, 0)),
        out_shape=jax.ShapeDtypeStruct((N_NODES, TROWS * 8), jnp.float32),
    )(zf, perm)


def _log16(x):
    bits = lax.bitcast_convert_type(x, jnp.int32)
    e = lax.shift_right_arithmetic(bits, 23) - 127
    mb = jnp.bitwise_or(jnp.bitwise_and(bits, 0x7FFFFF), 0x3F800000)
    m = lax.bitcast_convert_type(mb, jnp.float32)
    big = m > jnp.float32(1.4142135)
    m = jnp.where(big, m * jnp.float32(0.5), m)
    ef = e.astype(jnp.float32) + jnp.where(big, jnp.float32(1.0), jnp.float32(0.0))
    t = (m - jnp.float32(1.0)) / (m + jnp.float32(1.0))
    t2 = t * t
    p = t * (jnp.float32(2.0) + t2 * (jnp.float32(2.0 / 3.0)
         + t2 * (jnp.float32(0.4) + t2 * jnp.float32(2.0 / 7.0))))
    return ef * jnp.float32(0.6931471805599453) + p


def _rsqrt16(x):
    i = lax.bitcast_convert_type(x, jnp.int32)
    i = jnp.int32(0x5F3759DF) - lax.shift_right_arithmetic(i, 1)
    y = lax.bitcast_convert_type(i, jnp.float32)
    for _ in range(3):
        y = y * (jnp.float32(1.5) - jnp.float32(0.5) * x * y * y)
    return y


def _sc_body(t4_hbm, ts_hbm, s_hbm, r_hbm, nodes_hbm, out_hbm,
             s_b0, s_b1, r_b0, r_b1, ts_b0, ts_b1, d_b0, d_b1,
             si_b0, si_b1, ri_b0, ri_b1, kp_b0, kp_b1,
             sr_b0, sr_b1, rr_b0, rr_b1,
             nidx_v, nidx8_v, nrow8_v, part_v,
             ld_sem0, ld_sem1, g_sem0, g_sem1, n_sem):
    wid = lax.axis_index("s") * 2 + lax.axis_index("c")
    iota = lax.iota(jnp.int32, 16)

    s_b, r_b, ts_b = (s_b0, s_b1), (r_b0, r_b1), (ts_b0, ts_b1)
    d_b, si_b, ri_b = (d_b0, d_b1), (si_b0, si_b1), (ri_b0, ri_b1)
    kp_b = (kp_b0, kp_b1)
    srow_b, rrow_b = (sr_b0, sr_b1), (rr_b0, rr_b1)
    ld_sems = (ld_sem0, ld_sem1)
    g_sems = (g_sem0, g_sem1)

    nbase = pl.multiple_of(wid * NODES_PER_W, NODES_PER_W)
    pltpu.sync_copy(nodes_hbm.at[pl.ds(nbase, NODES_PER_W)], nidx_v)
    for g in range(NODES_PER_W // 16):
        nv = nidx_v[pl.ds(g * 16, 16)]
        for j in range(TROWS):
            plsc.store_scatter(
                nidx8_v, [(g * 16 + iota) * TROWS + j], nv * TROWS + j)
    nh = pltpu.async_copy(t4_hbm.at[nidx8_v], nrow8_v, n_sem)

    ebase0 = pl.multiple_of(wid * EV_PER_W, EV_PER_W)

    def fire_ld(nominal, sl):
        off = pl.multiple_of(
            jnp.minimum(nominal, N_ENTRIES - CHUNK), 8)
        pltpu.async_copy(s_hbm.at[pl.ds(off, CHUNK)], s_b[sl], ld_sems[sl])
        pltpu.async_copy(r_hbm.at[pl.ds(off, CHUNK)], r_b[sl], ld_sems[sl])
        pltpu.async_copy(ts_hbm.at[pl.ds(off, CHUNK)], ts_b[sl], ld_sems[sl])

    for sl in (0, 1):
        fire_ld(ebase0 + sl * CHUNK, sl)

    def node_phase():
        nh.wait()

        def node_k(k, carry):
            pr1, pr2 = carry
            is_pair = k < N_CP - 1
            ja = jnp.minimum(lax.shift_right_logical(k, 1), TROWS - 1)
            ca = 2 * (k - 2 * ja)
            jb = jnp.minimum(lax.shift_right_logical(k + 1, 1), TROWS - 1)
            cb2 = 2 * (k + 1 - 2 * jb)
            ca_v = jnp.full((16,), 0, jnp.int32) + ca
            cb_v = jnp.full((16,), 0, jnp.int32) + cb2

            def body(g, c):
                s0, s1, pq, pc, p1, p2 = c
                nloc = g * 16 + iota
                rowa = nloc * TROWS + ja
                rowb = nloc * TROWS + jb
                a0 = plsc.load_gather(nrow8_v, [rowa, ca_v])
                a1 = plsc.load_gather(nrow8_v, [rowa, ca_v + 1])
                b0 = plsc.load_gather(nrow8_v, [rowb, cb_v])
                b1 = plsc.load_gather(nrow8_v, [rowb, cb_v + 1])
                qk = a0 * a0 + a1 * a1
                qn = b0 * b0 + b1 * b1
                cd = a0 * b0 + a1 * b1
                d0 = b0 - a0
                d1 = b1 - a1
                cs = cd * _rsqrt16(qk * qn) - jnp.float32(1.0)
                return (s0 + a0, s1 + a1, pq + qk, pc + cd,
                        p1 + d0 * d0 + d1 * d1, p2 + cs * cs)

            z = jnp.zeros((16,), jnp.float32)
            s0, s1, pq, pc, p1, p2 = lax.fori_loop(
                0, NODES_PER_W // 16, body, (z, z, z, z, z, z))
            part_v[pl.ds((P_S0 + k) * 16, 16)] = s0
            part_v[pl.ds((P_S1 + k) * 16, 16)] = s1
            part_v[pl.ds((P_PQ + k) * 16, 16)] = pq
            part_v[pl.ds((P_PC + k) * 16, 16)] = jnp.where(is_pair, pc, jnp.float32(0.0))
            pr1 = pr1 + jnp.where(is_pair, p1, jnp.float32(0.0))
            pr2 = pr2 + jnp.where(is_pair, p2, jnp.float32(0.0))
            return pr1, pr2

        z16 = jnp.zeros((16,), jnp.float32)
        pr1_tot, pr2_tot = lax.fori_loop(0, N_CP, node_k, (z16, z16))
        part_v[pl.ds(P_PR1 * 16, 16)] = pr1_tot
        part_v[pl.ds(P_PR2 * 16, 16)] = pr2_tot
        part_v[pl.ds(3 * 16, 16)] = z16

    seg = jnp.float32(_SEG)

    def phase_a(sl):
        ssl, rsl, tsl = s_b[sl], r_b[sl], ts_b[sl]
        dsl, sil, ril, kpl = d_b[sl], si_b[sl], ri_b[sl], kp_b[sl]

        def body(g, _):
            sv = ssl[pl.ds(g * 16, 16)]
            rv = rsl[pl.ds(g * 16, 16)]
            tv = tsl[pl.ds(g * 16, 16)]
            t = tv / seg
            kap = t.astype(jnp.int32)
            d = t - kap.astype(jnp.float32)
            jrow = lax.shift_right_logical(kap, 1)
            sil[pl.ds(g * 16, 16)] = sv * TROWS + jrow
            ril[pl.ds(g * 16, 16)] = rv * TROWS + jrow
            kpl[pl.ds(g * 16, 16)] = lax.shift_left(jnp.bitwise_and(kap, 1), 1)
            dsl[pl.ds(g * 16, 16)] = d
            return 0

        lax.fori_loop(0, CHUNK // 16, body, 0)

    def fire_gathers(sl):
        pltpu.async_copy(t4_hbm.at[si_b[sl]], srow_b[sl], g_sems[sl])
        pltpu.async_copy(t4_hbm.at[ri_b[sl]], rrow_b[sl], g_sems[sl])

    def drain_gathers(sl):
        pltpu.make_async_copy(t4_hbm.at[pl.ds(0, CHUNK)], srow_b[sl], g_sems[sl]).wait()
        pltpu.make_async_copy(t4_hbm.at[pl.ds(0, CHUNK)], rrow_b[sl], g_sems[sl]).wait()

    def drain_ld(sl):
        pltpu.make_async_copy(s_hbm.at[pl.ds(0, CHUNK)], s_b[sl], ld_sems[sl]).wait()
        pltpu.make_async_copy(r_hbm.at[pl.ds(0, CHUNK)], r_b[sl], ld_sems[sl]).wait()
        pltpu.make_async_copy(ts_hbm.at[pl.ds(0, CHUNK)], ts_b[sl], ld_sems[sl]).wait()

    def phase_c(ci, sl, acc):
        srs, rrs, dsl, kpl = srow_b[sl], rrow_b[sl], d_b[sl], kp_b[sl]
        cbase = ebase0 + ci * CHUNK

        def body(g, acc):
            row = g * 16 + iota
            cb = kpl[pl.ds(g * 16, 16)]
            sc0 = plsc.load_gather(srs, [row, cb])
            sc1 = plsc.load_gather(srs, [row, cb + 1])
            sn0 = plsc.load_gather(srs, [row, cb + 2])
            sn1 = plsc.load_gather(srs, [row, cb + 3])
            rc0 = plsc.load_gather(rrs, [row, cb])
            rc1 = plsc.load_gather(rrs, [row, cb + 1])
            rn0 = plsc.load_gather(rrs, [row, cb + 2])
            rn1 = plsc.load_gather(rrs, [row, cb + 3])
            d = dsl[pl.ds(g * 16, 16)]
            omd = jnp.float32(1.0) - d
            u0 = omd * sc0 + d * sn0
            u1 = omd * sc1 + d * sn1
            v0 = omd * rc0 + d * rn0
            v1 = omd * rc1 + d * rn1
            first = u0 * v0 + u1 * v1
            lg = _log16(first)
            glob = cbase + g * 16 + iota
            return acc + jnp.where(glob < N_ENTRIES, lg, jnp.float32(0.0))

        return lax.fori_loop(0, CHUNK // 16, body, acc)

    drain_ld(0)
    phase_a(0)
    fire_gathers(0)
    drain_ld(1)
    phase_a(1)
    fire_gathers(1)
    fire_ld(ebase0 + 2 * CHUNK, 0)
    fire_ld(ebase0 + 3 * CHUNK, 1)

    node_phase()

    def pipe_body(i2, acc):
        a = 2 * i2
        acc = phase_c(a, 0, drain_gathers(0) or acc)
        drain_ld(0)
        phase_a(0)
        fire_gathers(0)
        fire_ld(ebase0 + (a + 4) * CHUNK, 0)
        acc = phase_c(a + 1, 1, drain_gathers(1) or acc)
        drain_ld(1)
        phase_a(1)
        fire_gathers(1)
        fire_ld(ebase0 + (a + 5) * CHUNK, 1)
        return acc

    acc = lax.fori_loop(0, NCHUNK // 2 - 1, pipe_body,
                        jnp.zeros((16,), jnp.float32))
    acc = phase_c(NCHUNK - 2, 0, drain_gathers(0) or acc)
    acc = phase_c(NCHUNK - 1, 1, drain_gathers(1) or acc)
    drain_ld(0)
    drain_ld(1)

    part_v[pl.ds(P_LOG * 16, 16)] = acc
    pltpu.sync_copy(part_v, out_hbm.at[wid])


def _sc_call(t4, ts_p, s_p, r_p, nodes):
    mesh = plsc.VectorSubcoreMesh(core_axis_name="c", subcore_axis_name="s")
    f = functools.partial(
        pl.kernel,
        out_type=jax.ShapeDtypeStruct((NW, P_H * 16), jnp.float32),
        mesh=mesh,
        compiler_params=pltpu.CompilerParams(
            needs_layout_passes=False, use_tc_tiling_on_sc=False),
        scratch_types=[
            pltpu.VMEM((CHUNK,), jnp.int32),
            pltpu.VMEM((CHUNK,), jnp.int32),
            pltpu.VMEM((CHUNK,), jnp.int32),
            pltpu.VMEM((CHUNK,), jnp.int32),
            pltpu.VMEM((CHUNK,), jnp.float32),
            pltpu.VMEM((CHUNK,), jnp.float32),
            pltpu.VMEM((CHUNK,), jnp.float32),
            pltpu.VMEM((CHUNK,), jnp.float32),
            pltpu.VMEM((CHUNK,), jnp.int32),
            pltpu.VMEM((CHUNK,), jnp.int32),
            pltpu.VMEM((CHUNK,), jnp.int32),
            pltpu.VMEM((CHUNK,), jnp.int32),
            pltpu.VMEM((CHUNK,), jnp.int32),
            pltpu.VMEM((CHUNK,), jnp.int32),
            pltpu.VMEM((CHUNK, T4_W), jnp.float32),
            pltpu.VMEM((CHUNK, T4_W), jnp.float32),
            pltpu.VMEM((CHUNK, T4_W), jnp.float32),
            pltpu.VMEM((CHUNK, T4_W), jnp.float32),
            pltpu.VMEM((NODES_PER_W,), jnp.int32),
            pltpu.VMEM((NODES_PER_W * TROWS,), jnp.int32),
            pltpu.VMEM((NODES_PER_W * TROWS, 8), jnp.float32),
            pltpu.VMEM((P_H * 16,), jnp.float32),
            pltpu.SemaphoreType.DMA,
            pltpu.SemaphoreType.DMA,
            pltpu.SemaphoreType.DMA,
            pltpu.SemaphoreType.DMA,
            pltpu.SemaphoreType.DMA,
        ],
    )(_sc_body)
    return f(t4, ts_p, s_p, r_p, nodes)


def _fin_body(pp_ref, o_ref):
    S = jnp.sum(jnp.sum(pp_ref[...], axis=0), axis=-1)
    prior = (jnp.float32(PENALTY / (BATCH_NODES * 2 * (N_CP - 1))) * S[P_PR1]
             + jnp.float32(PENALTY) * S[P_PR2])
    integral = jnp.float32(0.0)
    for k in range(N_CP - 1):
        dss_k = S[P_S0 + k] * S[P_S0 + k] + S[P_S1 + k] * S[P_S1 + k]
        dss_n = S[P_S0 + k + 1] * S[P_S0 + k + 1] + S[P_S1 + k + 1] * S[P_S1 + k + 1]
        dcr = S[P_S0 + k] * S[P_S0 + k + 1] + S[P_S1 + k] * S[P_S1 + k + 1]
        sij = ((dss_k - S[P_PQ + k]) / 6 + (dss_n - S[P_PQ + k + 1]) / 6
               + (dcr - S[P_PC + k]) / 6)
        integral = integral + jnp.float32(_CP[k + 1] - _CP[k]) * sij
    o_ref[...] = jnp.broadcast_to(prior - S[P_LOG] + integral, (1, 1))


def _finish(partials):
    return pl.pallas_call(
        _fin_body,
        out_shape=jax.ShapeDtypeStruct((1, 1), jnp.float32),
    )(partials)


@jax.jit
def kernel(Z, timestamps, nodes, senders, receivers):
    zf = Z.reshape(N_NODES, 2 * N_CP)
    t4v = _prep(zf, jnp.asarray(_perm_matrix()))
    t4 = t4v.reshape(N_NODES * TROWS, 8)

    partials = _sc_call(t4, timestamps, senders.astype(jnp.int32),
                        receivers.astype(jnp.int32), nodes.astype(jnp.int32))
    return _finish(partials.reshape(NW, P_H, 16))[0, 0]


# speedup vs baseline: 1.0054x; 1.0054x over previous
import functools
import numpy as np
import jax
import jax.numpy as jnp
from jax import lax
from jax.experimental import pallas as pl
from jax.experimental.pallas import tpu as pltpu
from jax.experimental.pallas import tpu_sc as plsc

N_NODES = 50000
N_CP = 17
N_ENTRIES = 1000000
BATCH_NODES = 4096
PENALTY = 10.0
TIME_MAX = 100.0

NW = 32
CHUNK = 2048
NCHUNK = 16
EV_PER_W = CHUNK * NCHUNK
E_PAD = NW * EV_PER_W
NODES_PER_W = BATCH_NODES // NW
T4_W = 8
TROWS = 8
TN_W = 40

_CP = (np.arange(N_CP, dtype=np.float32) / np.float32(N_CP - 1)) * np.float32(TIME_MAX + 0.0001)
_SEG = float(_CP[1] - _CP[0])

P_LOG, P_PR1, P_PR2 = 0, 1, 2
P_S0, P_S1, P_PQ, P_PC = 4, 21, 38, 55
P_H = 72


def _perm_matrix():
    P = np.zeros((2 * N_CP, TROWS * 8), dtype=np.float32)
    for j in range(TROWS):
        for c in range(8):
            t = 2 * j + c // 2
            if t <= N_CP - 1:
                src_col = t if c % 2 == 0 else N_CP + t
                P[src_col, 8 * j + c] = 1.0
    return P


def _prep_body(z_ref, p_ref, t4_ref):
    ze = jnp.exp(z_ref[...])
    mask = jnp.int32(-65536)
    b = lax.bitcast_convert_type(ze, jnp.int32)
    hi = lax.bitcast_convert_type(jnp.bitwise_and(b, mask), jnp.float32)
    r = ze - hi
    rb = lax.bitcast_convert_type(r, jnp.int32)
    mid = lax.bitcast_convert_type(jnp.bitwise_and(rb, mask), jnp.float32)
    pb = p_ref[...].astype(jnp.bfloat16)
    acc = jnp.dot(hi.astype(jnp.bfloat16), pb, preferred_element_type=jnp.float32)
    acc = acc + jnp.dot(mid.astype(jnp.bfloat16), pb, preferred_element_type=jnp.float32)
    t4_ref[...] = acc


def _prep(zf, perm):
    blk = 10000
    grid = N_NODES // blk
    return pl.pallas_call(
        _prep_body,
        grid=(grid,),
        in_specs=[
            pl.BlockSpec((blk, 2 * N_CP), lambda i: (i, 0)),
            pl.BlockSpec((2 * N_CP, TROWS * 8), lambda i: (0, 0)),
        ],
        out_specs=pl.BlockSpec((blk, TROWS * 8), lambda i: (i, 0)),
        out_shape=jax.ShapeDtypeStruct((N_NODES, TROWS * 8), jnp.float32),
    )(zf, perm)


def _log16(x):
    bits = lax.bitcast_convert_type(x, jnp.int32)
    e = lax.shift_right_arithmetic(bits, 23) - 127
    mb = jnp.bitwise_or(jnp.bitwise_and(bits, 0x7FFFFF), 0x3F800000)
    m = lax.bitcast_convert_type(mb, jnp.float32)
    big = m > jnp.float32(1.4142135)
    m = jnp.where(big, m * jnp.float32(0.5), m)
    ef = e.astype(jnp.float32) + jnp.where(big, jnp.float32(1.0), jnp.float32(0.0))
    t = (m - jnp.float32(1.0)) / (m + jnp.float32(1.0))
    t2 = t * t
    p = t * (jnp.float32(2.0) + t2 * (jnp.float32(2.0 / 3.0)
         + t2 * (jnp.float32(0.4) + t2 * jnp.float32(2.0 / 7.0))))
    return ef * jnp.float32(0.6931471805599453) + p


def _rsqrt16(x):
    i = lax.bitcast_convert_type(x, jnp.int32)
    i = jnp.int32(0x5F3759DF) - lax.shift_right_arithmetic(i, 1)
    y = lax.bitcast_convert_type(i, jnp.float32)
    for _ in range(3):
        y = y * (jnp.float32(1.5) - jnp.float32(0.5) * x * y * y)
    return y


def _sc_body(t4_hbm, ts_hbm, s_hbm, r_hbm, nodes_hbm, out_hbm,
             s_b0, s_b1, r_b0, r_b1, ts_b0, ts_b1, d_b0, d_b1,
             si_b0, si_b1, ri_b0, ri_b1, kp_b0, kp_b1,
             sr_b0, sr_b1, rr_b0, rr_b1,
             nidx_v, nidx8_v, nrow8_v, part_v,
             ld_sem0, ld_sem1, g_sem0, g_sem1, n_sem):
    wid = lax.axis_index("s") * 2 + lax.axis_index("c")
    iota = lax.iota(jnp.int32, 16)

    s_b, r_b, ts_b = (s_b0, s_b1), (r_b0, r_b1), (ts_b0, ts_b1)
    d_b, si_b, ri_b = (d_b0, d_b1), (si_b0, si_b1), (ri_b0, ri_b1)
    kp_b = (kp_b0, kp_b1)
    srow_b, rrow_b = (sr_b0, sr_b1), (rr_b0, rr_b1)
    ld_sems = (ld_sem0, ld_sem1)
    g_sems = (g_sem0, g_sem1)

    nbase = pl.multiple_of(wid * NODES_PER_W, NODES_PER_W)
    pltpu.sync_copy(nodes_hbm.at[pl.ds(nbase, NODES_PER_W)], nidx_v)
    for g in range(NODES_PER_W // 16):
        nv = nidx_v[pl.ds(g * 16, 16)]
        for j in range(TROWS):
            plsc.store_scatter(
                nidx8_v, [(g * 16 + iota) * TROWS + j], nv * TROWS + j)
    nh = pltpu.async_copy(t4_hbm.at[nidx8_v], nrow8_v, n_sem)

    ebase0 = pl.multiple_of(wid * EV_PER_W, EV_PER_W)

    def fire_ld(nominal, sl):
        off = pl.multiple_of(
            jnp.minimum(nominal, N_ENTRIES - CHUNK), 8)
        pltpu.async_copy(s_hbm.at[pl.ds(off, CHUNK)], s_b[sl], ld_sems[sl])
        pltpu.async_copy(r_hbm.at[pl.ds(off, CHUNK)], r_b[sl], ld_sems[sl])
        pltpu.async_copy(ts_hbm.at[pl.ds(off, CHUNK)], ts_b[sl], ld_sems[sl])

    for sl in (0, 1):
        fire_ld(ebase0 + sl * CHUNK, sl)

    def node_phase():
        nh.wait()

        def node_k(k, carry):
            pr1, pr2 = carry
            is_pair = k < N_CP - 1
            ja = jnp.minimum(lax.shift_right_logical(k, 1), TROWS - 1)
            ca = 2 * (k - 2 * ja)
            jb = jnp.minimum(lax.shift_right_logical(k + 1, 1), TROWS - 1)
            cb2 = 2 * (k + 1 - 2 * jb)
            ca_v = jnp.full((16,), 0, jnp.int32) + ca
            cb_v = jnp.full((16,), 0, jnp.int32) + cb2

            def body(g, c):
                s0, s1, pq, pc, p1, p2 = c
                nloc = g * 16 + iota
                rowa = nloc * TROWS + ja
                rowb = nloc * TROWS + jb
                a0 = plsc.load_gather(nrow8_v, [rowa, ca_v])
                a1 = plsc.load_gather(nrow8_v, [rowa, ca_v + 1])
                b0 = plsc.load_gather(nrow8_v, [rowb, cb_v])
                b1 = plsc.load_gather(nrow8_v, [rowb, cb_v + 1])
                qk = a0 * a0 + a1 * a1
                qn = b0 * b0 + b1 * b1
                cd = a0 * b0 + a1 * b1
                d0 = b0 - a0
                d1 = b1 - a1
                cs = cd * _rsqrt16(qk * qn) - jnp.float32(1.0)
                return (s0 + a0, s1 + a1, pq + qk, pc + cd,
                        p1 + d0 * d0 + d1 * d1, p2 + cs * cs)

            z = jnp.zeros((16,), jnp.float32)
            s0, s1, pq, pc, p1, p2 = lax.fori_loop(
                0, NODES_PER_W // 16, body, (z, z, z, z, z, z))
            part_v[pl.ds((P_S0 + k) * 16, 16)] = s0
            part_v[pl.ds((P_S1 + k) * 16, 16)] = s1
            part_v[pl.ds((P_PQ + k) * 16, 16)] = pq
            part_v[pl.ds((P_PC + k) * 16, 16)] = jnp.where(is_pair, pc, jnp.float32(0.0))
            pr1 = pr1 + jnp.where(is_pair, p1, jnp.float32(0.0))
            pr2 = pr2 + jnp.where(is_pair, p2, jnp.float32(0.0))
            return pr1, pr2

        z16 = jnp.zeros((16,), jnp.float32)
        pr1_tot, pr2_tot = lax.fori_loop(0, N_CP, node_k, (z16, z16))
        part_v[pl.ds(P_PR1 * 16, 16)] = pr1_tot
        part_v[pl.ds(P_PR2 * 16, 16)] = pr2_tot
        part_v[pl.ds(3 * 16, 16)] = z16

    seg = jnp.float32(_SEG)

    def phase_a(sl):
        ssl, rsl, tsl = s_b[sl], r_b[sl], ts_b[sl]
        dsl, sil, ril, kpl = d_b[sl], si_b[sl], ri_b[sl], kp_b[sl]

        def body(g, _):
            sv = ssl[pl.ds(g * 16, 16)]
            rv = rsl[pl.ds(g * 16, 16)]
            tv = tsl[pl.ds(g * 16, 16)]
            t = tv / seg
            kap = t.astype(jnp.int32)
            d = t - kap.astype(jnp.float32)
            jrow = lax.shift_right_logical(kap, 1)
            sil[pl.ds(g * 16, 16)] = sv * TROWS + jrow
            ril[pl.ds(g * 16, 16)] = rv * TROWS + jrow
            kpl[pl.ds(g * 16, 16)] = lax.shift_left(jnp.bitwise_and(kap, 1), 1)
            dsl[pl.ds(g * 16, 16)] = d
            return 0

        lax.fori_loop(0, CHUNK // 16, body, 0)

    def fire_gathers(sl):
        pltpu.async_copy(t4_hbm.at[si_b[sl]], srow_b[sl], g_sems[sl])
        pltpu.async_copy(t4_hbm.at[ri_b[sl]], rrow_b[sl], g_sems[sl])

    def drain_gathers(sl):
        pltpu.make_async_copy(t4_hbm.at[pl.ds(0, CHUNK)], srow_b[sl], g_sems[sl]).wait()
        pltpu.make_async_copy(t4_hbm.at[pl.ds(0, CHUNK)], rrow_b[sl], g_sems[sl]).wait()

    def drain_ld(sl):
        pltpu.make_async_copy(s_hbm.at[pl.ds(0, CHUNK)], s_b[sl], ld_sems[sl]).wait()
        pltpu.make_async_copy(r_hbm.at[pl.ds(0, CHUNK)], r_b[sl], ld_sems[sl]).wait()
        pltpu.make_async_copy(ts_hbm.at[pl.ds(0, CHUNK)], ts_b[sl], ld_sems[sl]).wait()

    def phase_c(ci, sl, acc):
        srs, rrs, dsl, kpl = srow_b[sl], rrow_b[sl], d_b[sl], kp_b[sl]
        cbase = ebase0 + ci * CHUNK

        def body(g, acc):
            row = g * 16 + iota
            cb = kpl[pl.ds(g * 16, 16)]
            sc0 = plsc.load_gather(srs, [row, cb])
            sc1 = plsc.load_gather(srs, [row, cb + 1])
            sn0 = plsc.load_gather(srs, [row, cb + 2])
            sn1 = plsc.load_gather(srs, [row, cb + 3])
            rc0 = plsc.load_gather(rrs, [row, cb])
            rc1 = plsc.load_gather(rrs, [row, cb + 1])
            rn0 = plsc.load_gather(rrs, [row, cb + 2])
            rn1 = plsc.load_gather(rrs, [row, cb + 3])
            d = dsl[pl.ds(g * 16, 16)]
            omd = jnp.float32(1.0) - d
            u0 = omd * sc0 + d * sn0
            u1 = omd * sc1 + d * sn1
            v0 = omd * rc0 + d * rn0
            v1 = omd * rc1 + d * rn1
            first = u0 * v0 + u1 * v1
            lg = _log16(first)
            glob = cbase + g * 16 + iota
            return acc + jnp.where(glob < N_ENTRIES, lg, jnp.float32(0.0))

        return lax.fori_loop(0, CHUNK // 16, body, acc)

    drain_ld(0)
    phase_a(0)
    fire_gathers(0)
    drain_ld(1)
    phase_a(1)
    fire_gathers(1)
    fire_ld(ebase0 + 2 * CHUNK, 0)
    fire_ld(ebase0 + 3 * CHUNK, 1)

    node_phase()

    def pipe_body(i2, acc):
        a = 2 * i2
        acc = phase_c(a, 0, drain_gathers(0) or acc)
        drain_ld(0)
        phase_a(0)
        fire_gathers(0)
        fire_ld(ebase0 + (a + 4) * CHUNK, 0)
        acc = phase_c(a + 1, 1, drain_gathers(1) or acc)
        drain_ld(1)
        phase_a(1)
        fire_gathers(1)
        fire_ld(ebase0 + (a + 5) * CHUNK, 1)
        return acc

    acc = lax.fori_loop(0, NCHUNK // 2 - 1, pipe_body,
                        jnp.zeros((16,), jnp.float32))
    acc = phase_c(NCHUNK - 2, 0, drain_gathers(0) or acc)
    acc = phase_c(NCHUNK - 1, 1, drain_gathers(1) or acc)
    drain_ld(0)
    drain_ld(1)

    part_v[pl.ds(P_LOG * 16, 16)] = acc
    pltpu.sync_copy(part_v, out_hbm.at[wid])


def _sc_call(t4, ts_p, s_p, r_p, nodes):
    mesh = plsc.VectorSubcoreMesh(core_axis_name="c", subcore_axis_name="s")
    f = functools.partial(
        pl.kernel,
        out_type=jax.ShapeDtypeStruct((NW, P_H * 16), jnp.float32),
        mesh=mesh,
        compiler_params=pltpu.CompilerParams(
            needs_layout_passes=False, use_tc_tiling_on_sc=False),
        scratch_types=[
            pltpu.VMEM((CHUNK,), jnp.int32),
            pltpu.VMEM((CHUNK,), jnp.int32),
            pltpu.VMEM((CHUNK,), jnp.int32),
            pltpu.VMEM((CHUNK,), jnp.int32),
            pltpu.VMEM((CHUNK,), jnp.float32),
            pltpu.VMEM((CHUNK,), jnp.float32),
            pltpu.VMEM((CHUNK,), jnp.float32),
            pltpu.VMEM((CHUNK,), jnp.float32),
            pltpu.VMEM((CHUNK,), jnp.int32),
            pltpu.VMEM((CHUNK,), jnp.int32),
            pltpu.VMEM((CHUNK,), jnp.int32),
            pltpu.VMEM((CHUNK,), jnp.int32),
            pltpu.VMEM((CHUNK,), jnp.int32),
            pltpu.VMEM((CHUNK,), jnp.int32),
            pltpu.VMEM((CHUNK, T4_W), jnp.float32),
            pltpu.VMEM((CHUNK, T4_W), jnp.float32),
            pltpu.VMEM((CHUNK, T4_W), jnp.float32),
            pltpu.VMEM((CHUNK, T4_W), jnp.float32),
            pltpu.VMEM((NODES_PER_W,), jnp.int32),
            pltpu.VMEM((NODES_PER_W * TROWS,), jnp.int32),
            pltpu.VMEM((NODES_PER_W * TROWS, 8), jnp.float32),
            pltpu.VMEM((P_H * 16,), jnp.float32),
            pltpu.SemaphoreType.DMA,
            pltpu.SemaphoreType.DMA,
            pltpu.SemaphoreType.DMA,
            pltpu.SemaphoreType.DMA,
            pltpu.SemaphoreType.DMA,
        ],
    )(_sc_body)
    return f(t4, ts_p, s_p, r_p, nodes)


def _fin_body(pp_ref, o_ref):
    S = jnp.sum(jnp.sum(pp_ref[...], axis=0), axis=-1)
    prior = (jnp.float32(PENALTY / (BATCH_NODES * 2 * (N_CP - 1))) * S[P_PR1]
             + jnp.float32(PENALTY) * S[P_PR2])
    integral = jnp.float32(0.0)
    for k in range(N_CP - 1):
        dss_k = S[P_S0 + k] * S[P_S0 + k] + S[P_S1 + k] * S[P_S1 + k]
        dss_n = S[P_S0 + k + 1] * S[P_S0 + k + 1] + S[P_S1 + k + 1] * S[P_S1 + k + 1]
        dcr = S[P_S0 + k] * S[P_S0 + k + 1] + S[P_S1 + k] * S[P_S1 + k + 1]
        sij = ((dss_k - S[P_PQ + k]) / 6 + (dss_n - S[P_PQ + k + 1]) / 6
               + (dcr - S[P_PC + k]) / 6)
        integral = integral + jnp.float32(_CP[k + 1] - _CP[k]) * sij
    o_ref[...] = jnp.broadcast_to(prior - S[P_LOG] + integral, (1, 1))


def _finish(partials):
    return pl.pallas_call(
        _fin_body,
        out_shape=jax.ShapeDtypeStruct((1, 1), jnp.float32),
    )(partials)


@jax.jit
def kernel(Z, timestamps, nodes, senders, receivers):
    zf = Z.reshape(N_NODES, 2 * N_CP)
    t4v = _prep(zf, jnp.asarray(_perm_matrix()))
    t4 = t4v.reshape(N_NODES * TROWS, 8)

    partials = _sc_call(t4, timestamps, senders.astype(jnp.int32),
                        receivers.astype(jnp.int32), nodes.astype(jnp.int32))
    return _finish(partials.reshape(NW, P_H, 16))[0, 0]
